# baseline (speedup 1.0000x reference)
import functools

import jax
import jax.numpy as jnp
from jax import lax
from jax.experimental import pallas as pl
from jax.experimental.pallas import tpu as pltpu
from jax.experimental.pallas import tpu_sc as plsc


def _segment_sum_sc(messages, tgt_indices, B, E, D, N):
    NC, NS = 2, 16
    BATCHES_PER_CORE = B // NC
    EPS = 1024
    ROWS = 256
    FULL_LOADS = ((0, 0, ROWS), (1, 256, ROWS), (2, 512, ROWS), (0, 768, ROWS))
    LAST_LOADS = ((0, 0, ROWS), (1, 256, ROWS), (2, 512, 128))
    NBUF = 3
    NPAD = ((N + 8 * NS - 1) // (8 * NS)) * (8 * NS)
    ZROWS = NPAD // NS
    ZSUB = 8
    LAST_OROWS = N - (NS - 1) * ZROWS

    mesh = plsc.VectorSubcoreMesh(core_axis_name="c", subcore_axis_name="s")

    @functools.partial(
        pl.kernel,
        out_type=jax.ShapeDtypeStruct((B, N, D), jnp.float32),
        mesh=mesh,
        scratch_types=[
            *[pltpu.VMEM((ROWS, D), jnp.float32) for _ in range(NBUF)],
            *[pltpu.VMEM((1, ROWS), jnp.int32) for _ in range(NBUF)],
            pltpu.VMEM((ZSUB, D), jnp.float32),
            pltpu.VMEM_SHARED((NPAD, D), jnp.float32),
            pltpu.VMEM_SHARED((NPAD, D), jnp.float32),
            pltpu.SemaphoreType.DMA,
            *[pltpu.SemaphoreType.DMA for _ in range(NBUF)],
            *[pltpu.SemaphoreType.DMA for _ in range(NBUF)],
            pltpu.SemaphoreType.DMA,
            pltpu.SemaphoreType.DMA,
            pltpu.SemaphoreType.DMA,
        ],
    )
    def k(msg_hbm, tgt_hbm, out_hbm, *rest):
        msg_vs = rest[:NBUF]
        idx_vs = rest[NBUF : 2 * NBUF]
        zeros_v = rest[2 * NBUF]
        accs = rest[2 * NBUF + 1 : 2 * NBUF + 3]
        isem = rest[2 * NBUF + 3]
        msems = rest[2 * NBUF + 4 : 2 * NBUF + 4 + NBUF]
        ssems = rest[2 * NBUF + 4 + NBUF : 2 * NBUF + 4 + 2 * NBUF]
        zsem, osem0, osem1 = rest[2 * NBUF + 4 + 2 * NBUF :]
        osems = (osem0, osem1)
        c = lax.axis_index("c")
        s = lax.axis_index("s")

        def out_copy(b, p, rows):
            r0 = s * ZROWS
            return pltpu.make_async_copy(
                accs[p].at[pl.ds(r0, rows)], out_hbm.at[b, pl.ds(r0, rows)], osems[p]
            )

        def out_start(b, p):
            @pl.when(s < NS - 1)
            def _():
                out_copy(b, p, ZROWS).start()

            @pl.when(s == NS - 1)
            def _():
                out_copy(b, p, LAST_OROWS).start()

        def out_wait(p):
            @pl.when(s < NS - 1)
            def _():
                out_copy(0, p, ZROWS).wait()

            @pl.when(s == NS - 1)
            def _():
                out_copy(0, p, LAST_OROWS).wait()

        @pl.loop(0, ZSUB)
        def _(r):
            @pl.loop(0, D, step=16)
            def _(col):
                zeros_v[r, pl.ds(col, 16)] = jnp.zeros((16,), jnp.float32)

        @pl.loop(0, BATCHES_PER_CORE, step=2)
        def _(bi0):
            for p in range(2):
                bi = bi0 + p
                b = c * BATCHES_PER_CORE + bi
                acc = accs[p]
                ebase = s * EPS

                def msg_copy(loads, li):
                    buf, eoff, rows = loads[li]
                    return pltpu.make_async_copy(
                        msg_hbm.at[b, pl.ds(ebase + eoff, rows)],
                        msg_vs[buf].at[pl.ds(0, rows)],
                        msems[buf],
                    )

                def idx_copy(loads, li):
                    buf, eoff, rows = loads[li]
                    return pltpu.make_async_copy(
                        tgt_hbm.at[b, pl.ds(ebase + eoff, rows)],
                        idx_vs[buf].at[0, pl.ds(0, rows)],
                        isem,
                    )

                def scatter_desc(loads, li):
                    buf, eoff, rows = loads[li]
                    return pltpu.make_async_copy(
                        msg_vs[buf].at[pl.ds(0, rows)],
                        acc.at[idx_vs[buf].at[0, pl.ds(0, rows)]],
                        ssems[buf],
                    )

                def prefetch(loads):
                    for li in range(NBUF):
                        msg_copy(loads, li).start()
                        idx_copy(loads, li).start()

                def scatter_start(loads, li):
                    idx_copy(loads, li).wait()
                    msg_copy(loads, li).wait()
                    scatter_desc(loads, li).start(add=True)

                def scatter_phase_full(loads):
                    for li in range(NBUF):
                        scatter_start(loads, li)
                    scatter_desc(loads, 0).wait()
                    msg_copy(loads, 3).start()
                    idx_copy(loads, 3).start()
                    scatter_start(loads, 3)
                    for li in range(NBUF):
                        scatter_desc(loads, li if li else 3).wait()

                def scatter_phase_last(loads):
                    for li in range(NBUF):
                        scatter_start(loads, li)
                    for li in range(NBUF):
                        scatter_desc(loads, li).wait()

                @pl.when(bi >= 2)
                def _():
                    out_wait(p)

                @pl.when(s < NS - 1)
                def _():
                    prefetch(FULL_LOADS)

                @pl.when(s == NS - 1)
                def _():
                    prefetch(LAST_LOADS)

                for z in range(ZROWS // ZSUB):
                    pltpu.make_async_copy(
                        zeros_v, acc.at[pl.ds(s * ZROWS + z * ZSUB, ZSUB)], zsem
                    ).start()

                for z in range(ZROWS // ZSUB):
                    pltpu.make_async_copy(
                        zeros_v, acc.at[pl.ds(s * ZROWS + z * ZSUB, ZSUB)], zsem
                    ).wait()

                plsc.subcore_barrier()

                @pl.when(s < NS - 1)
                def _():
                    scatter_phase_full(FULL_LOADS)

                @pl.when(s == NS - 1)
                def _():
                    scatter_phase_last(LAST_LOADS)

                plsc.subcore_barrier()

                out_start(b, p)

        out_wait(0)
        out_wait(1)

    return k(messages, tgt_indices)


@jax.jit
def kernel(messages, tgt_indices, atom_features_ref):
    B, E, D = messages.shape
    N = atom_features_ref.shape[1]
    return _segment_sum_sc(messages, tgt_indices, B, E, D, N)


# speedup vs baseline: 1.0697x; 1.0697x over previous
import functools

import jax
import jax.numpy as jnp
from jax import lax
from jax.experimental import pallas as pl
from jax.experimental.pallas import tpu as pltpu
from jax.experimental.pallas import tpu_sc as plsc


def _segment_sum_sc(messages, tgt_indices, B, E, D, N):
    NC, NS = 2, 16
    BATCHES_PER_CORE = B // NC
    EPS = 1024
    ROWS = 128
    NBUF = 4
    FULL_LOADS = tuple((li % NBUF, li * ROWS, ROWS) for li in range(EPS // ROWS))
    LAST_LOADS = tuple((li % NBUF, li * ROWS, ROWS) for li in range(5))
    NPAD = ((N + 8 * NS - 1) // (8 * NS)) * (8 * NS)
    ZROWS = NPAD // NS
    ZSUB = 8
    LAST_OROWS = N - (NS - 1) * ZROWS

    mesh = plsc.VectorSubcoreMesh(core_axis_name="c", subcore_axis_name="s")

    @functools.partial(
        pl.kernel,
        out_type=jax.ShapeDtypeStruct((B, N, D), jnp.float32),
        mesh=mesh,
        scratch_types=[
            *[pltpu.VMEM((ROWS, D), jnp.float32) for _ in range(NBUF)],
            *[pltpu.VMEM((1, ROWS), jnp.int32) for _ in range(NBUF)],
            pltpu.VMEM((ZSUB, D), jnp.float32),
            pltpu.VMEM_SHARED((NPAD, D), jnp.float32),
            pltpu.VMEM_SHARED((NPAD, D), jnp.float32),
            *[pltpu.SemaphoreType.DMA for _ in range(NBUF)],
            *[pltpu.SemaphoreType.DMA for _ in range(NBUF)],
            *[pltpu.SemaphoreType.DMA for _ in range(NBUF)],
            pltpu.SemaphoreType.DMA,
            pltpu.SemaphoreType.DMA,
            pltpu.SemaphoreType.DMA,
        ],
    )
    def k(msg_hbm, tgt_hbm, out_hbm, *rest):
        msg_vs = rest[:NBUF]
        idx_vs = rest[NBUF : 2 * NBUF]
        zeros_v = rest[2 * NBUF]
        accs = rest[2 * NBUF + 1 : 2 * NBUF + 3]
        isems = rest[2 * NBUF + 3 : 3 * NBUF + 3]
        msems = rest[3 * NBUF + 3 : 4 * NBUF + 3]
        ssems = rest[4 * NBUF + 3 : 5 * NBUF + 3]
        zsem, osem0, osem1 = rest[5 * NBUF + 3 :]
        osems = (osem0, osem1)
        c = lax.axis_index("c")
        s = lax.axis_index("s")
        b0 = c * BATCHES_PER_CORE

        def out_copy(b, p, rows):
            r0 = s * ZROWS
            return pltpu.make_async_copy(
                accs[p].at[pl.ds(r0, rows)], out_hbm.at[b, pl.ds(r0, rows)], osems[p]
            )

        def out_start(b, p):
            @pl.when(s < NS - 1)
            def _():
                out_copy(b, p, ZROWS).start()

            @pl.when(s == NS - 1)
            def _():
                out_copy(b, p, LAST_OROWS).start()

        def out_wait(p):
            @pl.when(s < NS - 1)
            def _():
                out_copy(0, p, ZROWS).wait()

            @pl.when(s == NS - 1)
            def _():
                out_copy(0, p, LAST_OROWS).wait()

        def msg_copy(b, loads, li):
            buf, eoff, rows = loads[li]
            ebase = s * EPS
            return pltpu.make_async_copy(
                msg_hbm.at[b, pl.ds(ebase + eoff, rows)],
                msg_vs[buf].at[pl.ds(0, rows)],
                msems[buf],
            )

        def idx_ref(loads, li):
            buf, _, rows = loads[li]
            return idx_vs[buf].at[0]

        def idx_copy(b, loads, li):
            buf, eoff, rows = loads[li]
            ebase = s * EPS
            return pltpu.make_async_copy(
                tgt_hbm.at[b, pl.ds(ebase + eoff, rows)],
                idx_ref(loads, li),
                isems[buf],
            )

        def scatter_desc(acc, loads, li):
            buf, eoff, rows = loads[li]
            return pltpu.make_async_copy(
                msg_vs[buf].at[pl.ds(0, rows)],
                acc.at[idx_ref(loads, li)],
                ssems[buf],
            )

        def prefetch(b, loads):
            for li in range(min(NBUF, len(loads))):
                msg_copy(b, loads, li).start()
                idx_copy(b, loads, li).start()

        def scatter_phase(b, acc, loads, more):
            n = len(loads)
            for li in range(n):
                idx_copy(b, loads, li).wait()
                msg_copy(b, loads, li).wait()
                scatter_desc(acc, loads, li).start(add=True)
                scatter_desc(acc, loads, li).wait()
                if li + NBUF < n:
                    msg_copy(b, loads, li + NBUF).start()
                    idx_copy(b, loads, li + NBUF).start()
            @pl.when(more)
            def _():
                prefetch(b + 1, loads)

        @pl.loop(0, ZSUB)
        def _(r):
            @pl.loop(0, D, step=16)
            def _(col):
                zeros_v[r, pl.ds(col, 16)] = jnp.zeros((16,), jnp.float32)

        @pl.when(s < NS - 1)
        def _():
            prefetch(b0, FULL_LOADS)

        @pl.when(s == NS - 1)
        def _():
            prefetch(b0, LAST_LOADS)

        @pl.loop(0, BATCHES_PER_CORE, step=2)
        def _(bi0):
            for p in range(2):
                bi = bi0 + p
                b = b0 + bi
                acc = accs[p]
                more = bi < BATCHES_PER_CORE - 1

                @pl.when(bi >= 2)
                def _():
                    out_wait(p)

                for z in range(ZROWS // ZSUB):
                    pltpu.make_async_copy(
                        zeros_v, acc.at[pl.ds(s * ZROWS + z * ZSUB, ZSUB)], zsem
                    ).start()

                for z in range(ZROWS // ZSUB):
                    pltpu.make_async_copy(
                        zeros_v, acc.at[pl.ds(s * ZROWS + z * ZSUB, ZSUB)], zsem
                    ).wait()

                plsc.subcore_barrier()

                @pl.when(s < NS - 1)
                def _():
                    scatter_phase(b, acc, FULL_LOADS, more)

                @pl.when(s == NS - 1)
                def _():
                    scatter_phase(b, acc, LAST_LOADS, more)

                plsc.subcore_barrier()

                out_start(b, p)

        out_wait(0)
        out_wait(1)

    return k(messages, tgt_indices)


@jax.jit
def kernel(messages, tgt_indices, atom_features_ref):
    B, E, D = messages.shape
    N = atom_features_ref.shape[1]
    return _segment_sum_sc(messages, tgt_indices, B, E, D, N)
